# baseline (speedup 1.0000x reference)
import functools

import jax
import jax.numpy as jnp
import numpy as np
from jax.experimental import pallas as pl
from jax.experimental.pallas import tpu as pltpu

N = 10000
RATIO = 0.25
R2 = 0.4 * 0.4
K = 64
M = 2500
MP = 2560
NB_A = MP // 128
NREST = N - MP
NRESTP = 7680
NB_B = NRESTP // 128


def _fps_kernel(pt_ref, sel_ref, dists_ref):
    lane = jax.lax.broadcasted_iota(jnp.int32, (1, 128), 1)
    flat = (jax.lax.broadcasted_iota(jnp.int32, (80, 128), 0) * 128
            + jax.lax.broadcasted_iota(jnp.int32, (80, 128), 1))

    def dist_to(r, l):
        d = None
        for c in range(6):
            row = pt_ref[c, pl.ds(r, 1), :]
            s = jnp.sum(jnp.where(lane == l, row, 0.0))
            dc = (pt_ref[c] - s) ** 2
            d = dc if d is None else d + dc
        return jnp.where(flat < N, d, -1.0)

    dists_ref[...] = dist_to(0, 0)
    sel_ref[0:1, 0:1] = jnp.zeros((1, 1), jnp.int32)

    def body(i, _):
        dists = dists_ref[...]
        mx = jnp.max(dists)
        cand = jnp.where(dists == mx, flat, jnp.int32(2 ** 30))
        nxt = jnp.min(cand)
        sel_ref[pl.ds(i, 1), :] = jnp.full((1, 1), nxt, jnp.int32)
        dists_ref[...] = jnp.minimum(dists, dist_to(nxt // 128, nxt % 128))
        return 0

    jax.lax.fori_loop(1, M, body, 0)


def _fps(pos, m):
    pt = jnp.pad(pos.T, ((0, 0), (0, 10240 - N))).reshape(6, 80, 128)
    sel = pl.pallas_call(
        _fps_kernel,
        out_shape=jax.ShapeDtypeStruct((MP, 1), jnp.int32),
        scratch_shapes=[pltpu.VMEM((80, 128), jnp.float32)],
    )(pt)
    return sel[:m, 0]


def _conv_attn_kernel(posn_ref, hn_ref, ssn_ref, valid_ref, sd_ref, ss_ref,
                      hi_ref, posi_ref, wp1_ref, bp1_ref, wp2_ref, bp2_ref,
                      wa_ref, ba_ref, wno_ref, wnp_ref, bn_ref, out_ref):
    b = pl.program_id(0)
    rows = 128
    e = rows * K
    posn3 = jnp.reshape(posn_ref[...], (rows, K, 8))
    posi = posi_ref[...]
    rel = jnp.reshape(posi[:, None, :] - posn3, (e, 8))
    t1 = jnp.maximum(jnp.dot(rel, wp1_ref[...],
                             preferred_element_type=jnp.float32)
                     + bp1_ref[...], 0.0)
    delta = jnp.maximum(jnp.dot(t1, wp2_ref[...],
                                preferred_element_type=jnp.float32)
                        + bp2_ref[...], 0.0)
    delta3 = jnp.reshape(delta, (rows, K, 128))
    wa_row = wa_ref[...]
    dwa = jnp.sum(delta3 * wa_row[None, :, :], axis=2)

    t10 = jnp.maximum(bp1_ref[...], 0.0)
    delta0 = jnp.maximum(jnp.dot(t10, wp2_ref[...],
                                 preferred_element_type=jnp.float32)
                         + bp2_ref[...], 0.0)
    dwa0 = jnp.sum(delta0 * wa_row, axis=1, keepdims=True)

    sd = sd_ref[...]
    ba = ba_ref[...]
    valid = valid_ref[...] > 0.0
    alphae = sd - ssn_ref[...] + dwa + ba
    alphae = jnp.where(valid, alphae, -1e30)
    alpha_self = sd - ss_ref[...] + dwa0 + ba
    m = jnp.maximum(jnp.max(alphae, axis=1, keepdims=True), alpha_self)
    pe = jnp.where(valid, jnp.exp(alphae - m), 0.0)
    ps = jnp.exp(alpha_self - m)
    den = jnp.sum(pe, axis=1, keepdims=True) + ps + 1e-16

    hn3 = jnp.reshape(hn_ref[...], (rows, K, 128))
    msg = jnp.sum(pe[:, :, None] * (hn3 + delta3), axis=1)
    msg = msg + ps * (hi_ref[...] + delta0)
    out_rows = msg / den

    hf = (jnp.dot(out_rows, wno_ref[...], preferred_element_type=jnp.float32)
          + jnp.dot(posi, wnp_ref[...], preferred_element_type=jnp.float32)
          + bn_ref[...])
    bmax = jnp.max(hf, axis=0, keepdims=True)

    @pl.when(b == 0)
    def _():
        out_ref[...] = bmax

    @pl.when(b > 0)
    def _():
        out_ref[...] = jnp.maximum(out_ref[...], bmax)


def _selfloop_kernel(x_ref, pos_ref, wlin_ref, wp2_ref, bp1_ref, bp2_ref,
                     wno_ref, wnp_ref, bn_ref, maxa_ref, out_ref):
    b = pl.program_id(0)
    h = jnp.dot(x_ref[...], wlin_ref[...],
                preferred_element_type=jnp.float32)
    t10 = jnp.maximum(bp1_ref[...], 0.0)
    delta0 = jnp.maximum(jnp.dot(t10, wp2_ref[...],
                                 preferred_element_type=jnp.float32)
                         + bp2_ref[...], 0.0)
    out_rows = h + delta0
    hf = (jnp.dot(out_rows, wno_ref[...], preferred_element_type=jnp.float32)
          + jnp.dot(pos_ref[...], wnp_ref[...],
                    preferred_element_type=jnp.float32)
          + bn_ref[...])
    row = MP + b * 128 + jax.lax.broadcasted_iota(jnp.int32, (128, 1), 0)
    hf = jnp.where(row < N, hf, -1e30)
    bmax = jnp.max(hf, axis=0, keepdims=True)

    @pl.when(b == 0)
    def _():
        out_ref[...] = jnp.maximum(maxa_ref[...], bmax)

    @pl.when(b > 0)
    def _():
        out_ref[...] = jnp.maximum(out_ref[...], bmax)


def kernel(x, pos, batch, W_lin, W_src, W_dst, W_p1, b_p1, W_p2, b_p2,
           W_a, b_a, W_n2, b_n2):
    idx = _fps(pos, M)
    q = pos[idx]
    d2 = (jnp.sum(q * q, 1)[:, None] + jnp.sum(pos * pos, 1)[None, :]
          - 2.0 * (q @ pos.T))
    nchunk = 16
    cw = N // nchunk
    negs, nbrs = jax.lax.top_k(-d2.reshape(M * nchunk, cw), K)
    negs = negs.reshape(M, nchunk, K)
    nbrs = (nbrs.reshape(M, nchunk, K)
            + (jnp.arange(nchunk, dtype=jnp.int32) * cw)[None, :, None])
    negc = negs.reshape(M, nchunk * K)
    nbrc = nbrs.reshape(M, nchunk * K)
    neg, pos_in_c = jax.lax.top_k(negc, K)
    nbr = jnp.take_along_axis(nbrc, pos_in_c, axis=1)
    valid = (-neg) <= R2

    h = x @ W_lin.T
    wa = W_a[0]
    ss = x @ (W_src.T @ wa)
    sd = x @ (W_dst.T @ wa)

    nbr_p = jnp.zeros((MP, K), jnp.int32).at[:M].set(nbr)
    valid_p = jnp.zeros((MP, K), jnp.float32).at[:M].set(
        valid.astype(jnp.float32))
    hn_flat = h[nbr_p.reshape(-1)]
    posn = pos[nbr_p.reshape(-1)]
    posn = jnp.pad(posn, ((0, 0), (0, 2)))
    ssn = ss[nbr_p]
    sd_col = sd[:MP][:, None]
    ss_col = ss[:MP][:, None]
    hi = h[:MP]
    posi = jnp.pad(pos[:MP], ((0, 0), (0, 2)))

    wp1 = jnp.pad(W_p1.T, ((0, 2), (0, 0)))
    wp2 = W_p2.T
    bp1 = b_p1[None, :]
    bp2 = b_p2[None, :]
    waT = wa[None, :]
    baM = b_a[None, :]
    wno = W_n2[:, :128].T
    wnp = jnp.pad(W_n2[:, 128:].T, ((0, 2), (0, 0)))
    bn = b_n2[None, :]

    full = lambda r, c: pl.BlockSpec((r, c), lambda b: (0, 0))
    maxa = pl.pallas_call(
        _conv_attn_kernel,
        grid=(NB_A,),
        in_specs=[
            pl.BlockSpec((128 * K, 8), lambda b: (b, 0)),
            pl.BlockSpec((128 * K, 128), lambda b: (b, 0)),
            pl.BlockSpec((128, K), lambda b: (b, 0)),
            pl.BlockSpec((128, K), lambda b: (b, 0)),
            pl.BlockSpec((128, 1), lambda b: (b, 0)),
            pl.BlockSpec((128, 1), lambda b: (b, 0)),
            pl.BlockSpec((128, 128), lambda b: (b, 0)),
            pl.BlockSpec((128, 8), lambda b: (b, 0)),
            full(8, 64), full(1, 64), full(64, 128), full(1, 128),
            full(1, 128), full(1, 1), full(128, 128), full(8, 128),
            full(1, 128),
        ],
        out_specs=pl.BlockSpec((1, 128), lambda b: (0, 0)),
        out_shape=jax.ShapeDtypeStruct((1, 128), jnp.float32),
    )(posn, hn_flat, ssn, valid_p, sd_col, ss_col, hi, posi,
      wp1, bp1, wp2, bp2, waT, baM, wno, wnp, bn)

    x2 = jnp.zeros((NRESTP, 128), jnp.float32).at[:NREST].set(x[MP:])
    pos2 = jnp.zeros((NRESTP, 8), jnp.float32).at[:NREST, :6].set(pos[MP:])
    pooled = pl.pallas_call(
        _selfloop_kernel,
        grid=(NB_B,),
        in_specs=[
            pl.BlockSpec((128, 128), lambda b: (b, 0)),
            pl.BlockSpec((128, 8), lambda b: (b, 0)),
            full(128, 128), full(64, 128), full(1, 64), full(1, 128),
            full(128, 128), full(8, 128), full(1, 128),
            full(1, 128),
        ],
        out_specs=pl.BlockSpec((1, 128), lambda b: (0, 0)),
        out_shape=jax.ShapeDtypeStruct((1, 128), jnp.float32),
    )(x2, pos2, W_lin.T, wp2, bp1, bp2, wno, wnp, bn, maxa)

    pos_out = jnp.zeros((1, 6), dtype=pos.dtype)
    batch_out = jnp.arange(1, dtype=jnp.int32)
    return (pooled, pos_out, batch_out)


# speedup vs baseline: 1.0447x; 1.0447x over previous
import functools

import jax
import jax.numpy as jnp
import numpy as np
from jax.experimental import pallas as pl
from jax.experimental.pallas import tpu as pltpu

N = 10000
RATIO = 0.25
R2 = 0.4 * 0.4
K = 64
M = 2500
MP = 2560
NB_A = MP // 128
NREST = N - MP
NRESTP = 7680
NB_B = NRESTP // 128


def _fps_kernel(pt_ref, sel_ref, dists_ref):
    lane = jax.lax.broadcasted_iota(jnp.int32, (1, 128), 1)
    flat = (jax.lax.broadcasted_iota(jnp.int32, (80, 128), 0) * 128
            + jax.lax.broadcasted_iota(jnp.int32, (80, 128), 1))

    def dist_to(r, l):
        d = None
        for c in range(6):
            row = pt_ref[c, pl.ds(r, 1), :]
            s = jnp.sum(jnp.where(lane == l, row, 0.0))
            dc = (pt_ref[c] - s) ** 2
            d = dc if d is None else d + dc
        return jnp.where(flat < N, d, -1.0)

    dists_ref[...] = dist_to(0, 0)
    sel_ref[0:1, 0:1] = jnp.zeros((1, 1), jnp.int32)

    def body(i, _):
        dists = dists_ref[...]
        mx = jnp.max(dists)
        cand = jnp.where(dists == mx, flat, jnp.int32(2 ** 30))
        nxt = jnp.min(cand)
        sel_ref[pl.ds(i, 1), :] = jnp.full((1, 1), nxt, jnp.int32)
        dists_ref[...] = jnp.minimum(dists, dist_to(nxt // 128, nxt % 128))
        return 0

    jax.lax.fori_loop(1, M, body, 0)


def _fps(pos, m):
    pt = jnp.pad(pos.T, ((0, 0), (0, 10240 - N))).reshape(6, 80, 128)
    sel = pl.pallas_call(
        _fps_kernel,
        out_shape=jax.ShapeDtypeStruct((MP, 1), jnp.int32),
        scratch_shapes=[pltpu.VMEM((80, 128), jnp.float32)],
    )(pt)
    return sel[:m, 0]


def _conv_attn_kernel(posn_ref, hn_ref, ssn_ref, valid_ref, sd_ref, ss_ref,
                      hi_ref, posi_ref, wp1_ref, bp1_ref, wp2_ref, bp2_ref,
                      wa_ref, ba_ref, wno_ref, wnp_ref, bn_ref, out_ref):
    b = pl.program_id(0)
    rows = 128
    e = rows * K
    posn3 = jnp.reshape(posn_ref[...], (rows, K, 8))
    posi = posi_ref[...]
    rel = jnp.reshape(posi[:, None, :] - posn3, (e, 8))
    t1 = jnp.maximum(jnp.dot(rel, wp1_ref[...],
                             preferred_element_type=jnp.float32)
                     + bp1_ref[...], 0.0)
    delta = jnp.maximum(jnp.dot(t1, wp2_ref[...],
                                preferred_element_type=jnp.float32)
                        + bp2_ref[...], 0.0)
    delta3 = jnp.reshape(delta, (rows, K, 128))
    wa_row = wa_ref[...]
    dwa = jnp.sum(delta3 * wa_row[None, :, :], axis=2)

    t10 = jnp.maximum(bp1_ref[...], 0.0)
    delta0 = jnp.maximum(jnp.dot(t10, wp2_ref[...],
                                 preferred_element_type=jnp.float32)
                         + bp2_ref[...], 0.0)
    dwa0 = jnp.sum(delta0 * wa_row, axis=1, keepdims=True)

    sd = sd_ref[...]
    ba = ba_ref[...]
    valid = valid_ref[...] > 0.0
    alphae = sd - ssn_ref[...] + dwa + ba
    alphae = jnp.where(valid, alphae, -1e30)
    alpha_self = sd - ss_ref[...] + dwa0 + ba
    m = jnp.maximum(jnp.max(alphae, axis=1, keepdims=True), alpha_self)
    pe = jnp.where(valid, jnp.exp(alphae - m), 0.0)
    ps = jnp.exp(alpha_self - m)
    den = jnp.sum(pe, axis=1, keepdims=True) + ps + 1e-16

    hn3 = jnp.reshape(hn_ref[...], (rows, K, 128))
    msg = jnp.sum(pe[:, :, None] * (hn3 + delta3), axis=1)
    msg = msg + ps * (hi_ref[...] + delta0)
    out_rows = msg / den

    hf = (jnp.dot(out_rows, wno_ref[...], preferred_element_type=jnp.float32)
          + jnp.dot(posi, wnp_ref[...], preferred_element_type=jnp.float32)
          + bn_ref[...])
    bmax = jnp.max(hf, axis=0, keepdims=True)

    @pl.when(b == 0)
    def _():
        out_ref[...] = bmax

    @pl.when(b > 0)
    def _():
        out_ref[...] = jnp.maximum(out_ref[...], bmax)


def _selfloop_kernel(x_ref, pos_ref, wlin_ref, wp2_ref, bp1_ref, bp2_ref,
                     wno_ref, wnp_ref, bn_ref, maxa_ref, out_ref):
    b = pl.program_id(0)
    h = jnp.dot(x_ref[...], wlin_ref[...],
                preferred_element_type=jnp.float32)
    t10 = jnp.maximum(bp1_ref[...], 0.0)
    delta0 = jnp.maximum(jnp.dot(t10, wp2_ref[...],
                                 preferred_element_type=jnp.float32)
                         + bp2_ref[...], 0.0)
    out_rows = h + delta0
    hf = (jnp.dot(out_rows, wno_ref[...], preferred_element_type=jnp.float32)
          + jnp.dot(pos_ref[...], wnp_ref[...],
                    preferred_element_type=jnp.float32)
          + bn_ref[...])
    row = MP + b * 128 + jax.lax.broadcasted_iota(jnp.int32, (128, 1), 0)
    hf = jnp.where(row < N, hf, -1e30)
    bmax = jnp.max(hf, axis=0, keepdims=True)

    @pl.when(b == 0)
    def _():
        out_ref[...] = jnp.maximum(maxa_ref[...], bmax)

    @pl.when(b > 0)
    def _():
        out_ref[...] = jnp.maximum(out_ref[...], bmax)


def kernel(x, pos, batch, W_lin, W_src, W_dst, W_p1, b_p1, W_p2, b_p2,
           W_a, b_a, W_n2, b_n2):
    idx = _fps(pos, M)
    q = pos[idx]
    d2 = (jnp.sum(q * q, 1)[:, None] + jnp.sum(pos * pos, 1)[None, :]
          - 2.0 * (q @ pos.T))
    nchunk = 4
    cw = N // nchunk
    negs, nbrs = jax.lax.top_k(-d2.reshape(M * nchunk, cw), K)
    negs = negs.reshape(M, nchunk, K)
    nbrs = (nbrs.reshape(M, nchunk, K)
            + (jnp.arange(nchunk, dtype=jnp.int32) * cw)[None, :, None])
    negc = negs.reshape(M, nchunk * K)
    nbrc = nbrs.reshape(M, nchunk * K)
    neg, pos_in_c = jax.lax.top_k(negc, K)
    nbr = jnp.take_along_axis(nbrc, pos_in_c, axis=1)
    valid = (-neg) <= R2

    h = x @ W_lin.T
    wa = W_a[0]
    ss = x @ (W_src.T @ wa)
    sd = x @ (W_dst.T @ wa)

    nbr_p = jnp.zeros((MP, K), jnp.int32).at[:M].set(nbr)
    valid_p = jnp.zeros((MP, K), jnp.float32).at[:M].set(
        valid.astype(jnp.float32))
    hn_flat = h[nbr_p.reshape(-1)]
    posn = pos[nbr_p.reshape(-1)]
    posn = jnp.pad(posn, ((0, 0), (0, 2)))
    ssn = ss[nbr_p]
    sd_col = sd[:MP][:, None]
    ss_col = ss[:MP][:, None]
    hi = h[:MP]
    posi = jnp.pad(pos[:MP], ((0, 0), (0, 2)))

    wp1 = jnp.pad(W_p1.T, ((0, 2), (0, 0)))
    wp2 = W_p2.T
    bp1 = b_p1[None, :]
    bp2 = b_p2[None, :]
    waT = wa[None, :]
    baM = b_a[None, :]
    wno = W_n2[:, :128].T
    wnp = jnp.pad(W_n2[:, 128:].T, ((0, 2), (0, 0)))
    bn = b_n2[None, :]

    full = lambda r, c: pl.BlockSpec((r, c), lambda b: (0, 0))
    maxa = pl.pallas_call(
        _conv_attn_kernel,
        grid=(NB_A,),
        in_specs=[
            pl.BlockSpec((128 * K, 8), lambda b: (b, 0)),
            pl.BlockSpec((128 * K, 128), lambda b: (b, 0)),
            pl.BlockSpec((128, K), lambda b: (b, 0)),
            pl.BlockSpec((128, K), lambda b: (b, 0)),
            pl.BlockSpec((128, 1), lambda b: (b, 0)),
            pl.BlockSpec((128, 1), lambda b: (b, 0)),
            pl.BlockSpec((128, 128), lambda b: (b, 0)),
            pl.BlockSpec((128, 8), lambda b: (b, 0)),
            full(8, 64), full(1, 64), full(64, 128), full(1, 128),
            full(1, 128), full(1, 1), full(128, 128), full(8, 128),
            full(1, 128),
        ],
        out_specs=pl.BlockSpec((1, 128), lambda b: (0, 0)),
        out_shape=jax.ShapeDtypeStruct((1, 128), jnp.float32),
    )(posn, hn_flat, ssn, valid_p, sd_col, ss_col, hi, posi,
      wp1, bp1, wp2, bp2, waT, baM, wno, wnp, bn)

    x2 = jnp.zeros((NRESTP, 128), jnp.float32).at[:NREST].set(x[MP:])
    pos2 = jnp.zeros((NRESTP, 8), jnp.float32).at[:NREST, :6].set(pos[MP:])
    pooled = pl.pallas_call(
        _selfloop_kernel,
        grid=(NB_B,),
        in_specs=[
            pl.BlockSpec((128, 128), lambda b: (b, 0)),
            pl.BlockSpec((128, 8), lambda b: (b, 0)),
            full(128, 128), full(64, 128), full(1, 64), full(1, 128),
            full(128, 128), full(8, 128), full(1, 128),
            full(1, 128),
        ],
        out_specs=pl.BlockSpec((1, 128), lambda b: (0, 0)),
        out_shape=jax.ShapeDtypeStruct((1, 128), jnp.float32),
    )(x2, pos2, W_lin.T, wp2, bp1, bp2, wno, wnp, bn, maxa)

    pos_out = jnp.zeros((1, 6), dtype=pos.dtype)
    batch_out = jnp.arange(1, dtype=jnp.int32)
    return (pooled, pos_out, batch_out)


# speedup vs baseline: 1.0741x; 1.0281x over previous
import functools

import jax
import jax.numpy as jnp
import numpy as np
from jax.experimental import pallas as pl
from jax.experimental.pallas import tpu as pltpu

N = 10000
RATIO = 0.25
R2 = 0.4 * 0.4
K = 64
M = 2500
MP = 2560
NB_A = MP // 128
NREST = N - MP
NRESTP = 7680
NB_B = NRESTP // 128


def _fps_kernel(pt_ref, sel_ref, dists_ref):
    lane = jax.lax.broadcasted_iota(jnp.int32, (1, 128), 1)
    flat = (jax.lax.broadcasted_iota(jnp.int32, (80, 128), 0) * 128
            + jax.lax.broadcasted_iota(jnp.int32, (80, 128), 1))

    def dist_to(r, l):
        d = None
        for c in range(6):
            row = pt_ref[c, pl.ds(r, 1), :]
            s = jnp.sum(jnp.where(lane == l, row, 0.0))
            dc = (pt_ref[c] - s) ** 2
            d = dc if d is None else d + dc
        return jnp.where(flat < N, d, -1.0)

    dists_ref[...] = dist_to(0, 0)
    sel_ref[0:1, 0:1] = jnp.zeros((1, 1), jnp.int32)

    def body(i, _):
        dists = dists_ref[...]
        mx = jnp.max(dists)
        cand = jnp.where(dists == mx, flat, jnp.int32(2 ** 30))
        nxt = jnp.min(cand)
        sel_ref[pl.ds(i, 1), :] = jnp.full((1, 1), nxt, jnp.int32)
        dists_ref[...] = jnp.minimum(dists, dist_to(nxt // 128, nxt % 128))
        return 0

    jax.lax.fori_loop(1, M, body, 0)


def _fps(pos, m):
    pt = jnp.pad(pos.T, ((0, 0), (0, 10240 - N))).reshape(6, 80, 128)
    sel = pl.pallas_call(
        _fps_kernel,
        out_shape=jax.ShapeDtypeStruct((MP, 1), jnp.int32),
        scratch_shapes=[pltpu.VMEM((80, 128), jnp.float32)],
    )(pt)
    return sel[:m, 0]


def _conv_attn_kernel(posn_ref, hn_ref, ssn_ref, valid_ref, sd_ref, ss_ref,
                      hi_ref, posi_ref, wp1_ref, bp1_ref, wp2_ref, bp2_ref,
                      wa_ref, ba_ref, wno_ref, wnp_ref, bn_ref, out_ref):
    b = pl.program_id(0)
    rows = 128
    e = rows * K
    posn3 = jnp.reshape(posn_ref[...], (rows, K, 8))
    posi = posi_ref[...]
    rel = jnp.reshape(posi[:, None, :] - posn3, (e, 8))
    t1 = jnp.maximum(jnp.dot(rel, wp1_ref[...],
                             preferred_element_type=jnp.float32)
                     + bp1_ref[...], 0.0)
    delta = jnp.maximum(jnp.dot(t1, wp2_ref[...],
                                preferred_element_type=jnp.float32)
                        + bp2_ref[...], 0.0)
    delta3 = jnp.reshape(delta, (rows, K, 128))
    wa_row = wa_ref[...]
    dwa = jnp.sum(delta3 * wa_row[None, :, :], axis=2)

    t10 = jnp.maximum(bp1_ref[...], 0.0)
    delta0 = jnp.maximum(jnp.dot(t10, wp2_ref[...],
                                 preferred_element_type=jnp.float32)
                         + bp2_ref[...], 0.0)
    dwa0 = jnp.sum(delta0 * wa_row, axis=1, keepdims=True)

    sd = sd_ref[...]
    ba = ba_ref[...]
    valid = valid_ref[...] > 0.0
    alphae = sd - ssn_ref[...] + dwa + ba
    alphae = jnp.where(valid, alphae, -1e30)
    alpha_self = sd - ss_ref[...] + dwa0 + ba
    m = jnp.maximum(jnp.max(alphae, axis=1, keepdims=True), alpha_self)
    pe = jnp.where(valid, jnp.exp(alphae - m), 0.0)
    ps = jnp.exp(alpha_self - m)
    den = jnp.sum(pe, axis=1, keepdims=True) + ps + 1e-16

    hn3 = jnp.reshape(hn_ref[...], (rows, K, 128))
    msg = jnp.sum(pe[:, :, None] * (hn3 + delta3), axis=1)
    msg = msg + ps * (hi_ref[...] + delta0)
    out_rows = msg / den

    hf = (jnp.dot(out_rows, wno_ref[...], preferred_element_type=jnp.float32)
          + jnp.dot(posi, wnp_ref[...], preferred_element_type=jnp.float32)
          + bn_ref[...])
    bmax = jnp.max(hf, axis=0, keepdims=True)

    @pl.when(b == 0)
    def _():
        out_ref[...] = bmax

    @pl.when(b > 0)
    def _():
        out_ref[...] = jnp.maximum(out_ref[...], bmax)


def _selfloop_kernel(x_ref, pos_ref, wlin_ref, wp2_ref, bp1_ref, bp2_ref,
                     wno_ref, wnp_ref, bn_ref, maxa_ref, out_ref):
    b = pl.program_id(0)
    h = jnp.dot(x_ref[...], wlin_ref[...],
                preferred_element_type=jnp.float32)
    t10 = jnp.maximum(bp1_ref[...], 0.0)
    delta0 = jnp.maximum(jnp.dot(t10, wp2_ref[...],
                                 preferred_element_type=jnp.float32)
                         + bp2_ref[...], 0.0)
    out_rows = h + delta0
    hf = (jnp.dot(out_rows, wno_ref[...], preferred_element_type=jnp.float32)
          + jnp.dot(pos_ref[...], wnp_ref[...],
                    preferred_element_type=jnp.float32)
          + bn_ref[...])
    row = MP + b * 128 + jax.lax.broadcasted_iota(jnp.int32, (128, 1), 0)
    hf = jnp.where(row < N, hf, -1e30)
    bmax = jnp.max(hf, axis=0, keepdims=True)

    @pl.when(b == 0)
    def _():
        out_ref[...] = jnp.maximum(maxa_ref[...], bmax)

    @pl.when(b > 0)
    def _():
        out_ref[...] = jnp.maximum(out_ref[...], bmax)


def kernel(x, pos, batch, W_lin, W_src, W_dst, W_p1, b_p1, W_p2, b_p2,
           W_a, b_a, W_n2, b_n2):
    idx = _fps(pos, M)
    q = pos[idx]
    d2 = (jnp.sum(q * q, 1)[:, None] + jnp.sum(pos * pos, 1)[None, :]
          - 2.0 * (q @ pos.T))
    nchunk = 8
    cw = N // nchunk
    negs, nbrs = jax.lax.top_k(-d2.reshape(M * nchunk, cw), K)
    negs = negs.reshape(M, nchunk, K)
    nbrs = (nbrs.reshape(M, nchunk, K)
            + (jnp.arange(nchunk, dtype=jnp.int32) * cw)[None, :, None])
    negc = negs.reshape(M, nchunk * K)
    nbrc = nbrs.reshape(M, nchunk * K)
    neg, pos_in_c = jax.lax.top_k(negc, K)
    nbr = jnp.take_along_axis(nbrc, pos_in_c, axis=1)
    valid = (-neg) <= R2

    h = x @ W_lin.T
    wa = W_a[0]
    ss = x @ (W_src.T @ wa)
    sd = x @ (W_dst.T @ wa)

    nbr_p = jnp.zeros((MP, K), jnp.int32).at[:M].set(nbr)
    valid_p = jnp.zeros((MP, K), jnp.float32).at[:M].set(
        valid.astype(jnp.float32))
    hn_flat = h[nbr_p.reshape(-1)]
    posn = pos[nbr_p.reshape(-1)]
    posn = jnp.pad(posn, ((0, 0), (0, 2)))
    ssn = ss[nbr_p]
    sd_col = sd[:MP][:, None]
    ss_col = ss[:MP][:, None]
    hi = h[:MP]
    posi = jnp.pad(pos[:MP], ((0, 0), (0, 2)))

    wp1 = jnp.pad(W_p1.T, ((0, 2), (0, 0)))
    wp2 = W_p2.T
    bp1 = b_p1[None, :]
    bp2 = b_p2[None, :]
    waT = wa[None, :]
    baM = b_a[None, :]
    wno = W_n2[:, :128].T
    wnp = jnp.pad(W_n2[:, 128:].T, ((0, 2), (0, 0)))
    bn = b_n2[None, :]

    full = lambda r, c: pl.BlockSpec((r, c), lambda b: (0, 0))
    maxa = pl.pallas_call(
        _conv_attn_kernel,
        grid=(NB_A,),
        in_specs=[
            pl.BlockSpec((128 * K, 8), lambda b: (b, 0)),
            pl.BlockSpec((128 * K, 128), lambda b: (b, 0)),
            pl.BlockSpec((128, K), lambda b: (b, 0)),
            pl.BlockSpec((128, K), lambda b: (b, 0)),
            pl.BlockSpec((128, 1), lambda b: (b, 0)),
            pl.BlockSpec((128, 1), lambda b: (b, 0)),
            pl.BlockSpec((128, 128), lambda b: (b, 0)),
            pl.BlockSpec((128, 8), lambda b: (b, 0)),
            full(8, 64), full(1, 64), full(64, 128), full(1, 128),
            full(1, 128), full(1, 1), full(128, 128), full(8, 128),
            full(1, 128),
        ],
        out_specs=pl.BlockSpec((1, 128), lambda b: (0, 0)),
        out_shape=jax.ShapeDtypeStruct((1, 128), jnp.float32),
    )(posn, hn_flat, ssn, valid_p, sd_col, ss_col, hi, posi,
      wp1, bp1, wp2, bp2, waT, baM, wno, wnp, bn)

    x2 = jnp.zeros((NRESTP, 128), jnp.float32).at[:NREST].set(x[MP:])
    pos2 = jnp.zeros((NRESTP, 8), jnp.float32).at[:NREST, :6].set(pos[MP:])
    pooled = pl.pallas_call(
        _selfloop_kernel,
        grid=(NB_B,),
        in_specs=[
            pl.BlockSpec((128, 128), lambda b: (b, 0)),
            pl.BlockSpec((128, 8), lambda b: (b, 0)),
            full(128, 128), full(64, 128), full(1, 64), full(1, 128),
            full(128, 128), full(8, 128), full(1, 128),
            full(1, 128),
        ],
        out_specs=pl.BlockSpec((1, 128), lambda b: (0, 0)),
        out_shape=jax.ShapeDtypeStruct((1, 128), jnp.float32),
    )(x2, pos2, W_lin.T, wp2, bp1, bp2, wno, wnp, bn, maxa)

    pos_out = jnp.zeros((1, 6), dtype=pos.dtype)
    batch_out = jnp.arange(1, dtype=jnp.int32)
    return (pooled, pos_out, batch_out)
